# baseline (speedup 1.0000x reference)
import functools

import jax
import jax.numpy as jnp
from jax import lax
from jax.experimental import pallas as pl
from jax.experimental.pallas import tpu as pltpu
from jax.experimental.pallas import tpu_sc as plsc

VOCAB = 1000
DIM = 2
WINDOW = 5
BATCH = 16384

_LANES = 16


def _make_sc_gather():
  info = plsc.get_sparse_core_info()
  nc, ns = info.num_cores, info.num_subcores
  nw = nc * ns
  b_per_w = BATCH // nw
  words_per_w = b_per_w * WINDOW * DIM
  mesh = plsc.VectorSubcoreMesh(core_axis_name="c", subcore_axis_name="s")

  @functools.partial(
      pl.kernel,
      mesh=mesh,
      compiler_params=pltpu.CompilerParams(needs_layout_passes=False),
      out_type=jax.ShapeDtypeStruct((BATCH * WINDOW * DIM,), jnp.float32),
      scratch_types=[
          pltpu.VMEM((b_per_w,), jnp.int32),
          pltpu.VMEM((VOCAB * DIM,), jnp.float32),
          pltpu.VMEM((words_per_w,), jnp.float32),
      ],
  )
  def sc_gather(x_hbm, tab_hbm, e5_hbm, x_v, tab_v, e5_v):
    wid = lax.axis_index("s") * nc + lax.axis_index("c")
    base = wid * b_per_w
    pltpu.sync_copy(x_hbm.at[pl.ds(base, b_per_w)], x_v)
    pltpu.sync_copy(tab_hbm, tab_v)
    lanes = lax.iota(jnp.int32, _LANES)
    for i in range(b_per_w // _LANES):
      idx = x_v[pl.ds(i * _LANES, _LANES)]
      f0 = plsc.load_gather(tab_v, [idx * DIM])
      f1 = plsc.load_gather(tab_v, [idx * DIM + 1])
      dst = (lanes + i * _LANES) * (WINDOW * DIM)
      for w in range(WINDOW):
        plsc.store_scatter(e5_v, [dst + 2 * w], f0)
        plsc.store_scatter(e5_v, [dst + 2 * w + 1], f1)
    pltpu.sync_copy(e5_v, e5_hbm.at[pl.ds(wid * words_per_w, words_per_w)])

  return sc_gather


_sc_gather = _make_sc_gather()


def _tc_project(e5_ref, wt_ref, b_ref, out_ref):
  e = e5_ref[...]
  w0 = wt_ref[0:1, :]
  w1 = wt_ref[1:2, :]
  bias = b_ref[...]
  out_ref[...] = e[:, 0:1] * w0 + e[:, 1:2] * w1 + bias


_BB = 2048


def kernel(x, emb_table, W, b):
  rows = BATCH * WINDOW
  e5 = _sc_gather(x.astype(jnp.int32), emb_table.reshape(-1))
  e5 = e5.reshape(rows, DIM)
  wt = W.T
  b2 = b.reshape(1, VOCAB)
  out2 = pl.pallas_call(
      _tc_project,
      grid=(rows // _BB,),
      in_specs=[
          pl.BlockSpec((_BB, DIM), lambda i: (i, 0)),
          pl.BlockSpec((DIM, VOCAB), lambda i: (0, 0)),
          pl.BlockSpec((1, VOCAB), lambda i: (0, 0)),
      ],
      out_specs=pl.BlockSpec((_BB, VOCAB), lambda i: (i, 0)),
      out_shape=jax.ShapeDtypeStruct((rows, VOCAB), jnp.float32),
  )(e5, wt, b2)
  return out2.reshape(BATCH, WINDOW, VOCAB)


# speedup vs baseline: 1.3880x; 1.3880x over previous
import functools

import jax
import jax.numpy as jnp
from jax import lax
from jax.experimental import pallas as pl
from jax.experimental.pallas import tpu as pltpu
from jax.experimental.pallas import tpu_sc as plsc

VOCAB = 1000
DIM = 2
WINDOW = 5
BATCH = 16384

_LANES = 16


def _make_sc_gather():
  info = plsc.get_sparse_core_info()
  nc, ns = info.num_cores, info.num_subcores
  nw = nc * ns
  b_per_w = BATCH // nw
  words_per_w = b_per_w * DIM
  mesh = plsc.VectorSubcoreMesh(core_axis_name="c", subcore_axis_name="s")

  @functools.partial(
      pl.kernel,
      mesh=mesh,
      compiler_params=pltpu.CompilerParams(needs_layout_passes=False),
      out_type=jax.ShapeDtypeStruct((BATCH * DIM,), jnp.float32),
      scratch_types=[
          pltpu.VMEM((b_per_w,), jnp.int32),
          pltpu.VMEM((VOCAB * DIM,), jnp.float32),
          pltpu.VMEM((words_per_w,), jnp.float32),
      ],
  )
  def sc_gather(x_hbm, tab_hbm, e_hbm, x_v, tab_v, e_v):
    wid = lax.axis_index("s") * nc + lax.axis_index("c")
    base = wid * b_per_w
    pltpu.sync_copy(x_hbm.at[pl.ds(base, b_per_w)], x_v)
    pltpu.sync_copy(tab_hbm, tab_v)
    lanes = lax.iota(jnp.int32, _LANES)
    for i in range(b_per_w // _LANES):
      idx = x_v[pl.ds(i * _LANES, _LANES)]
      f0 = plsc.load_gather(tab_v, [idx * DIM])
      f1 = plsc.load_gather(tab_v, [idx * DIM + 1])
      dst = (lanes + i * _LANES) * DIM
      plsc.store_scatter(e_v, [dst], f0)
      plsc.store_scatter(e_v, [dst + 1], f1)
    pltpu.sync_copy(e_v, e_hbm.at[pl.ds(wid * words_per_w, words_per_w)])

  return sc_gather


_sc_gather = _make_sc_gather()

_BB = 256


def _tc_project(e_ref, wt_ref, b_ref, out_ref):
  e = e_ref[...]
  logits = (e[:, 0:1] * wt_ref[0:1, :] + e[:, 1:2] * wt_ref[1:2, :]
            + b_ref[...])
  out_ref[...] = jnp.broadcast_to(logits[:, None, :], (_BB, WINDOW, VOCAB))


def kernel(x, emb_table, W, b):
  e = _sc_gather(x.astype(jnp.int32), emb_table.reshape(-1))
  e = e.reshape(BATCH, DIM)
  wt = W.T
  b2 = b.reshape(1, VOCAB)
  out = pl.pallas_call(
      _tc_project,
      grid=(BATCH // _BB,),
      in_specs=[
          pl.BlockSpec((_BB, DIM), lambda i: (i, 0)),
          pl.BlockSpec((DIM, VOCAB), lambda i: (0, 0)),
          pl.BlockSpec((1, VOCAB), lambda i: (0, 0)),
      ],
      out_specs=pl.BlockSpec((_BB, WINDOW, VOCAB), lambda i: (i, 0, 0)),
      out_shape=jax.ShapeDtypeStruct((BATCH, WINDOW, VOCAB), jnp.float32),
  )(e, wt, b2)
  return out


# speedup vs baseline: 6.6227x; 4.7715x over previous
import functools

import jax
import jax.numpy as jnp
from jax import lax
from jax.experimental import pallas as pl
from jax.experimental.pallas import tpu as pltpu
from jax.experimental.pallas import tpu_sc as plsc

VOCAB = 1000
DIM = 2
WINDOW = 5
BATCH = 16384

_LANES = 16


def _make_sc_gather():
  info = plsc.get_sparse_core_info()
  nc, ns = info.num_cores, info.num_subcores
  nw = nc * ns
  b_per_w = BATCH // nw
  mesh = plsc.VectorSubcoreMesh(core_axis_name="c", subcore_axis_name="s")

  @functools.partial(
      pl.kernel,
      mesh=mesh,
      compiler_params=pltpu.CompilerParams(needs_layout_passes=False),
      out_type=jax.ShapeDtypeStruct((DIM * BATCH,), jnp.float32),
      scratch_types=[
          pltpu.VMEM((b_per_w,), jnp.int32),
          pltpu.VMEM((VOCAB * DIM,), jnp.float32),
          pltpu.VMEM((b_per_w,), jnp.float32),
          pltpu.VMEM((b_per_w,), jnp.float32),
      ],
  )
  def sc_gather(x_hbm, tab_hbm, e_hbm, x_v, tab_v, e0_v, e1_v):
    wid = lax.axis_index("s") * nc + lax.axis_index("c")
    base = wid * b_per_w
    pltpu.sync_copy(x_hbm.at[pl.ds(base, b_per_w)], x_v)
    pltpu.sync_copy(tab_hbm, tab_v)
    for i in range(b_per_w // _LANES):
      idx = x_v[pl.ds(i * _LANES, _LANES)]
      e0_v[pl.ds(i * _LANES, _LANES)] = plsc.load_gather(tab_v, [idx * DIM])
      e1_v[pl.ds(i * _LANES, _LANES)] = plsc.load_gather(tab_v, [idx * DIM + 1])
    pltpu.sync_copy(e0_v, e_hbm.at[pl.ds(base, b_per_w)])
    pltpu.sync_copy(e1_v, e_hbm.at[pl.ds(BATCH + base, b_per_w)])

  return sc_gather


_sc_gather = _make_sc_gather()

_BBL = 1024


def _tc_project(e_ref, w_ref, b_ref, out_ref):
  logits = (w_ref[:, 0:1] * e_ref[0:1, :] + w_ref[:, 1:2] * e_ref[1:2, :]
            + b_ref[...])
  out_ref[...] = jnp.broadcast_to(logits[None], (WINDOW, VOCAB, _BBL))


def kernel(x, emb_table, W, b):
  e = _sc_gather(x.astype(jnp.int32), emb_table.reshape(-1))
  e = e.reshape(DIM, BATCH)
  bcol = b.reshape(VOCAB, 1)
  out_t = pl.pallas_call(
      _tc_project,
      grid=(BATCH // _BBL,),
      in_specs=[
          pl.BlockSpec((DIM, _BBL), lambda i: (0, i)),
          pl.BlockSpec((VOCAB, DIM), lambda i: (0, 0)),
          pl.BlockSpec((VOCAB, 1), lambda i: (0, 0)),
      ],
      out_specs=pl.BlockSpec((WINDOW, VOCAB, _BBL), lambda i: (0, 0, i)),
      out_shape=jax.ShapeDtypeStruct((WINDOW, VOCAB, BATCH), jnp.float32),
  )(e, W, bcol)
  return jnp.transpose(out_t, (2, 0, 1))


# speedup vs baseline: 6.7544x; 1.0199x over previous
import functools

import jax
import jax.numpy as jnp
from jax import lax
from jax.experimental import pallas as pl
from jax.experimental.pallas import tpu as pltpu
from jax.experimental.pallas import tpu_sc as plsc

VOCAB = 1000
DIM = 2
WINDOW = 5
BATCH = 16384

_LANES = 16


def _make_sc_gather():
  info = plsc.get_sparse_core_info()
  nc, ns = info.num_cores, info.num_subcores
  nw = nc * ns
  b_per_w = BATCH // nw
  mesh = plsc.VectorSubcoreMesh(core_axis_name="c", subcore_axis_name="s")

  @functools.partial(
      pl.kernel,
      mesh=mesh,
      compiler_params=pltpu.CompilerParams(needs_layout_passes=False),
      out_type=jax.ShapeDtypeStruct((DIM * BATCH,), jnp.float32),
      scratch_types=[
          pltpu.VMEM((b_per_w,), jnp.int32),
          pltpu.VMEM((VOCAB * DIM,), jnp.float32),
          pltpu.VMEM((b_per_w,), jnp.float32),
          pltpu.VMEM((b_per_w,), jnp.float32),
      ],
  )
  def sc_gather(x_hbm, tab_hbm, e_hbm, x_v, tab_v, e0_v, e1_v):
    wid = lax.axis_index("s") * nc + lax.axis_index("c")
    base = wid * b_per_w
    pltpu.sync_copy(x_hbm.at[pl.ds(base, b_per_w)], x_v)
    pltpu.sync_copy(tab_hbm, tab_v)
    for i in range(b_per_w // _LANES):
      idx = x_v[pl.ds(i * _LANES, _LANES)]
      e0_v[pl.ds(i * _LANES, _LANES)] = plsc.load_gather(tab_v, [idx * DIM])
      e1_v[pl.ds(i * _LANES, _LANES)] = plsc.load_gather(tab_v, [idx * DIM + 1])
    pltpu.sync_copy(e0_v, e_hbm.at[pl.ds(base, b_per_w)])
    pltpu.sync_copy(e1_v, e_hbm.at[pl.ds(BATCH + base, b_per_w)])

  return sc_gather


_sc_gather = _make_sc_gather()

_BV = 40


def _tc_project(e_ref, w_ref, b_ref, out_ref):
  logits = (w_ref[:, 0:1] * e_ref[0:1, :] + w_ref[:, 1:2] * e_ref[1:2, :]
            + b_ref[...])
  out_ref[...] = jnp.broadcast_to(logits[None], (WINDOW, _BV, BATCH))


def kernel(x, emb_table, W, b):
  e = _sc_gather(x.astype(jnp.int32), emb_table.reshape(-1))
  e = e.reshape(DIM, BATCH)
  bcol = b.reshape(VOCAB, 1)
  out_t = pl.pallas_call(
      _tc_project,
      grid=(VOCAB // _BV,),
      in_specs=[
          pl.BlockSpec((DIM, BATCH), lambda i: (0, 0)),
          pl.BlockSpec((_BV, DIM), lambda i: (i, 0)),
          pl.BlockSpec((_BV, 1), lambda i: (i, 0)),
      ],
      out_specs=pl.BlockSpec((WINDOW, _BV, BATCH), lambda i: (0, i, 0)),
      out_shape=jax.ShapeDtypeStruct((WINDOW, VOCAB, BATCH), jnp.float32),
  )(e, W, bcol)
  return jnp.transpose(out_t, (2, 0, 1))


# speedup vs baseline: 6.8841x; 1.0192x over previous
import functools

import jax
import jax.numpy as jnp
from jax import lax
from jax.experimental import pallas as pl
from jax.experimental.pallas import tpu as pltpu
from jax.experimental.pallas import tpu_sc as plsc

VOCAB = 1000
DIM = 2
WINDOW = 5
BATCH = 16384

_LANES = 16


def _make_sc_gather():
  info = plsc.get_sparse_core_info()
  nc, ns = info.num_cores, info.num_subcores
  nw = nc * ns
  b_per_w = BATCH // nw
  mesh = plsc.VectorSubcoreMesh(core_axis_name="c", subcore_axis_name="s")

  @functools.partial(
      pl.kernel,
      mesh=mesh,
      compiler_params=pltpu.CompilerParams(needs_layout_passes=False),
      out_type=jax.ShapeDtypeStruct((DIM * BATCH,), jnp.float32),
      scratch_types=[
          pltpu.VMEM((b_per_w,), jnp.int32),
          pltpu.VMEM((VOCAB * DIM,), jnp.float32),
          pltpu.VMEM((b_per_w,), jnp.float32),
          pltpu.VMEM((b_per_w,), jnp.float32),
      ],
  )
  def sc_gather(x_hbm, tab_hbm, e_hbm, x_v, tab_v, e0_v, e1_v):
    wid = lax.axis_index("s") * nc + lax.axis_index("c")
    base = wid * b_per_w
    pltpu.sync_copy(x_hbm.at[pl.ds(base, b_per_w)], x_v)
    pltpu.sync_copy(tab_hbm, tab_v)
    for i in range(b_per_w // _LANES):
      idx = x_v[pl.ds(i * _LANES, _LANES)]
      e0_v[pl.ds(i * _LANES, _LANES)] = plsc.load_gather(tab_v, [idx * DIM])
      e1_v[pl.ds(i * _LANES, _LANES)] = plsc.load_gather(tab_v, [idx * DIM + 1])
    pltpu.sync_copy(e0_v, e_hbm.at[pl.ds(base, b_per_w)])
    pltpu.sync_copy(e1_v, e_hbm.at[pl.ds(BATCH + base, b_per_w)])

  return sc_gather


_sc_gather = _make_sc_gather()

_BV = 40


def _tc_project(e_ref, w_ref, out_ref):
  e0 = e_ref[pl.ds(0, BATCH)].reshape(1, BATCH)
  e1 = e_ref[pl.ds(BATCH, BATCH)].reshape(1, BATCH)
  logits = (w_ref[:, 0:1] * e0 + w_ref[:, 1:2] * e1
            + w_ref[:, 2:3])
  out_ref[...] = jnp.broadcast_to(logits[None], (WINDOW, _BV, BATCH))


def kernel(x, emb_table, W, b):
  e = _sc_gather(x.astype(jnp.int32), emb_table.reshape(-1))
  waug = jnp.concatenate([W, b.reshape(VOCAB, 1)], axis=1)
  out_t = pl.pallas_call(
      _tc_project,
      grid=(VOCAB // _BV,),
      in_specs=[
          pl.BlockSpec((DIM * BATCH,), lambda i: (0,)),
          pl.BlockSpec((_BV, DIM + 1), lambda i: (i, 0)),
      ],
      out_specs=pl.BlockSpec((WINDOW, _BV, BATCH), lambda i: (0, i, 0)),
      out_shape=jax.ShapeDtypeStruct((WINDOW, VOCAB, BATCH), jnp.float32),
  )(e, waug)
  return jnp.transpose(out_t, (2, 0, 1))
